# Initial kernel scaffold; baseline (speedup 1.0000x reference)
import jax
import jax.numpy as jnp
from jax.experimental import pallas as pl


def kernel(point_cloud, query_cloud, temperature):
    raise NotImplementedError("write your pallas kernel here")


# speedup vs baseline: 27.6936x; 27.6936x over previous
import functools

import jax
import jax.numpy as jnp
from jax.experimental import pallas as pl
from jax.experimental.pallas import tpu as pltpu

GROUP_SIZE = 16
MIN_SIGMA = 1e-4
QB = 256


def _soft_projection_kernel(sig_ref, p_ref, q_ref, out_ref):
    inv_sigma = 1.0 / sig_ref[0]
    p = p_ref[0]
    q = q_ref[0]

    qq = jnp.sum(q * q, axis=0)
    pp = jnp.sum(p * p, axis=0)
    qp = jax.lax.dot_general(
        q.astype(jnp.bfloat16), p.astype(jnp.bfloat16),
        (((0,), (0,)), ((), ())),
        preferred_element_type=jnp.float32)
    d2_sel = qq[:, None] - 2.0 * qp + pp[None, :]

    d2 = None
    for c in range(3):
        diff = q[c][:, None] - p[c][None, :]
        term = diff * diff
        d2 = term if d2 is None else d2 + term

    big = jnp.float32(3.0e38)
    work = d2_sel
    thresh = None
    for _ in range(GROUP_SIZE):
        mn = jnp.min(work, axis=1, keepdims=True)
        thresh = mn
        work = jnp.where(work <= mn, big, work)

    row_min = jnp.min(d2, axis=1, keepdims=True)
    mask = d2_sel <= thresh
    w = jnp.where(mask, jnp.exp((row_min - d2) * inv_sigma), 0.0)
    z = jnp.sum(w, axis=1)

    proj = jax.lax.dot_general(
        p, w, (((1,), (1,)), ((), ())),
        preferred_element_type=jnp.float32)
    out_ref[0] = proj / z[None, :]


@jax.jit
def kernel(point_cloud, query_cloud, temperature):
    b, c, n = point_cloud.shape
    _, _, m = query_cloud.shape
    sigma = jnp.maximum(temperature * temperature, jnp.float32(MIN_SIGMA))
    sigma = jnp.reshape(sigma, (1,)).astype(jnp.float32)

    grid = (b, m // QB)
    return pl.pallas_call(
        _soft_projection_kernel,
        grid=grid,
        in_specs=[
            pl.BlockSpec(memory_space=pltpu.SMEM),
            pl.BlockSpec((1, c, n), lambda i, j: (i, 0, 0)),
            pl.BlockSpec((1, c, QB), lambda i, j: (i, 0, j)),
        ],
        out_specs=pl.BlockSpec((1, c, QB), lambda i, j: (i, 0, j)),
        out_shape=jax.ShapeDtypeStruct((b, c, m), jnp.float32),
    )(sigma, point_cloud, query_cloud)


# speedup vs baseline: 30.0564x; 1.0853x over previous
import functools

import jax
import jax.numpy as jnp
from jax.experimental import pallas as pl
from jax.experimental.pallas import tpu as pltpu

GROUP_SIZE = 16
MIN_SIGMA = 1e-4
QB = 256


def _soft_projection_kernel(sig_ref, p_ref, q_ref, out_ref):
    inv_sigma = 1.0 / sig_ref[0]
    p = p_ref[0]
    q = q_ref[0]

    qq = jnp.sum(q * q, axis=0)
    pp = jnp.sum(p * p, axis=0)
    qp = jax.lax.dot_general(
        q.astype(jnp.bfloat16), p.astype(jnp.bfloat16),
        (((0,), (0,)), ((), ())),
        preferred_element_type=jnp.float32)
    d2_sel = qq[:, None] - 2.0 * qp + pp[None, :]

    qp_acc = jax.lax.dot_general(
        q, p, (((0,), (0,)), ((), ())),
        precision=jax.lax.Precision.HIGHEST,
        preferred_element_type=jnp.float32)
    d2 = qq[:, None] - 2.0 * qp_acc + pp[None, :]

    big = jnp.float32(3.0e38)
    work = d2_sel
    row_min = None
    thresh = None
    for _ in range(GROUP_SIZE):
        mn = jnp.min(work, axis=1, keepdims=True)
        if row_min is None:
            row_min = mn
        thresh = mn
        work = jnp.where(work <= mn, big, work)

    mask = d2_sel <= thresh
    arg = jnp.minimum((row_min - d2) * inv_sigma, jnp.float32(80.0))
    w = jnp.where(mask, jnp.exp(arg), 0.0)
    z = jnp.sum(w, axis=1)

    proj = jax.lax.dot_general(
        p, w, (((1,), (1,)), ((), ())),
        preferred_element_type=jnp.float32)
    out_ref[0] = proj / z[None, :]


@jax.jit
def kernel(point_cloud, query_cloud, temperature):
    b, c, n = point_cloud.shape
    _, _, m = query_cloud.shape
    sigma = jnp.maximum(temperature * temperature, jnp.float32(MIN_SIGMA))
    sigma = jnp.reshape(sigma, (1,)).astype(jnp.float32)

    grid = (b, m // QB)
    return pl.pallas_call(
        _soft_projection_kernel,
        grid=grid,
        in_specs=[
            pl.BlockSpec(memory_space=pltpu.SMEM),
            pl.BlockSpec((1, c, n), lambda i, j: (i, 0, 0)),
            pl.BlockSpec((1, c, QB), lambda i, j: (i, 0, j)),
        ],
        out_specs=pl.BlockSpec((1, c, QB), lambda i, j: (i, 0, j)),
        out_shape=jax.ShapeDtypeStruct((b, c, m), jnp.float32),
    )(sigma, point_cloud, query_cloud)
